# Initial kernel scaffold; baseline (speedup 1.0000x reference)
import jax
import jax.numpy as jnp
from jax.experimental import pallas as pl


def kernel(h, adj, cond, y_fid, y_idx, lib_h, lib_adj, lib_freq, W_node1, W_msg1, W_read1, W_cond, w_term, b_term, W_node2, W_msg2, W_read2, W_idx0, W_idx1, w_idx2):
    raise NotImplementedError("write your pallas kernel here")


# speedup vs baseline: 1.3976x; 1.3976x over previous
import functools

import jax
import jax.numpy as jnp
from jax import lax
from jax.experimental import pallas as pl
from jax.experimental.pallas import tpu as pltpu
from jax.experimental.pallas import tpu_sc as plsc

N = 4096
V = 32
FIN = 64
FC = 16
D = 128
K = 10000
M = 16
NSAMPLE = 2
EPS = 1e-12

_KEYS = ((3625411723, 1954958720), (195045567, 4062205631))
_ROT = (13, 15, 26, 6, 17, 29, 16, 24)

_CH = 128
_NCH = 79
_KPAD = _CH * _NCH
_RB = 128
_TINY = 1.1754943508222875e-38


def _threefry_bits(key, idx):
    u32 = jnp.uint32
    k0 = u32(key[0])
    k1 = u32(key[1])
    k2 = u32(key[0] ^ key[1] ^ 0x1BD11BDA)
    ks = (k0, k1, k2)
    x0 = jnp.full_like(idx, k0)
    x1 = idx + k1
    for r in range(5):
        rr = _ROT[0:4] if r % 2 == 0 else _ROT[4:8]
        for j in range(4):
            x0 = x0 + x1
            x1 = ((x1 << u32(rr[j])) | (x1 >> u32(32 - rr[j]))) ^ x0
        x0 = x0 + ks[(r + 1) % 3]
        x1 = x1 + ks[(r + 2) % 3] + u32(r + 1)
    return x0 ^ x1


def _sample_body(freq_ref, yfid_ref, yn0_ref, yn1_ref, mv0, mi0, mv1, mi1):
    i = pl.program_id(0)
    c = pl.program_id(1)

    @pl.when(c == 0)
    def _init():
        mv0[...] = jnp.full((1, _RB), jnp.inf, jnp.float32)
        mi0[...] = jnp.zeros((1, _RB), jnp.int32)
        mv1[...] = jnp.full((1, _RB), jnp.inf, jnp.float32)
        mi1[...] = jnp.zeros((1, _RB), jnp.int32)

    yf = yfid_ref[0]
    sub = lax.broadcasted_iota(jnp.int32, (_CH, _RB), 0)
    lane = lax.broadcasted_iota(jnp.uint32, (_CH, _RB), 1)
    k_i = sub + c * _CH
    row = i.astype(jnp.uint32) * jnp.uint32(_RB) + lane
    idx = row * jnp.uint32(K) + k_i.astype(jnp.uint32)

    invf = 1.0 / freq_ref[0]
    valid = (k_i < K) & (k_i != yf)

    for key, mv, mi in ((_KEYS[0], mv0, mi0), (_KEYS[1], mv1, mi1)):
        bits = _threefry_bits(key, idx)
        u = lax.bitcast_convert_type(
            (bits >> jnp.uint32(9)) | jnp.uint32(0x3F800000), jnp.float32) - 1.0
        uf = jnp.maximum(jnp.float32(_TINY), u + jnp.float32(_TINY))
        s = -jnp.log(uf) * invf
        s = jnp.where(valid, s, jnp.inf)
        sv = jnp.min(s, axis=0, keepdims=True)
        si = jnp.min(jnp.where(s == sv, k_i, jnp.int32(1 << 30)),
                     axis=0, keepdims=True)
        upd = sv < mv[...]
        mv[...] = jnp.where(upd, sv, mv[...])
        mi[...] = jnp.where(upd, si, mi[...])

    @pl.when(c == _NCH - 1)
    def _emit():
        yn0_ref[...] = mi0[...].reshape(1, 1, _RB)
        yn1_ref[...] = mi1[...].reshape(1, 1, _RB)


def _sample_negatives(lib_freq, y_fid_c):
    freq = jnp.concatenate(
        [lib_freq, jnp.ones((_KPAD - K,), jnp.float32)]).reshape(_NCH, _CH, 1)
    yf = y_fid_c.reshape(N // _RB, 1, _RB)
    yn0, yn1 = pl.pallas_call(
        _sample_body,
        grid=(N // _RB, _NCH),
        in_specs=[
            pl.BlockSpec((1, _CH, 1), lambda i, c: (c, 0, 0)),
            pl.BlockSpec((1, 1, _RB), lambda i, c: (i, 0, 0)),
        ],
        out_specs=[
            pl.BlockSpec((1, 1, _RB), lambda i, c: (i, 0, 0)),
            pl.BlockSpec((1, 1, _RB), lambda i, c: (i, 0, 0)),
        ],
        out_shape=[
            jax.ShapeDtypeStruct((N // _RB, 1, _RB), jnp.int32),
            jax.ShapeDtypeStruct((N // _RB, 1, _RB), jnp.int32),
        ],
        scratch_shapes=[
            pltpu.VMEM((1, _RB), jnp.float32), pltpu.VMEM((1, _RB), jnp.int32),
            pltpu.VMEM((1, _RB), jnp.float32), pltpu.VMEM((1, _RB), jnp.int32),
        ],
    )(freq, yf)
    return yn0.reshape(N), yn1.reshape(N)


_LB = 400
_LPACK = 16


def _lib_body(h_ref, a_ref, wn_ref, wm_ref, wr_ref, out_ref, bd_ref, msg_ref):
    @pl.when(pl.program_id(0) == 0)
    def _z():
        bd_ref[...] = jnp.zeros((256, 256), jnp.float32)

    a = a_ref[...].astype(jnp.float32)
    h2 = h_ref[...].reshape(_LB * M, FIN)
    x = jnp.maximum(jnp.dot(h2, wn_ref[...], preferred_element_type=jnp.float32), 0.0)
    z = jnp.dot(x, wm_ref[...], preferred_element_type=jnp.float32)
    for p in range(_LB * M // 256):
        for g in range(_LPACK):
            bd_ref[M * g:M * (g + 1), M * g:M * (g + 1)] = a[_LPACK * p + g]
        msg_ref[256 * p:256 * (p + 1), :] = jnp.dot(
            bd_ref[...], z[256 * p:256 * (p + 1), :],
            preferred_element_type=jnp.float32)
    x2 = jnp.maximum(x + msg_ref[...], 0.0)
    gs = jnp.sum(x2.reshape(_LB, M, D), axis=1)
    out_ref[...] = jnp.maximum(
        jnp.dot(gs, wr_ref[...], preferred_element_type=jnp.float32), 0.0)


def _lib_embed(lib_h, lib_adj, wn, wm, wr):
    return pl.pallas_call(
        _lib_body,
        grid=(K // _LB,),
        in_specs=[
            pl.BlockSpec((_LB, M, FIN), lambda i: (i, 0, 0)),
            pl.BlockSpec((_LB, M, M), lambda i: (i, 0, 0)),
            pl.BlockSpec((FIN, D), lambda i: (0, 0)),
            pl.BlockSpec((D, D), lambda i: (0, 0)),
            pl.BlockSpec((D, D), lambda i: (0, 0)),
        ],
        out_specs=pl.BlockSpec((_LB, D), lambda i: (i, 0)),
        out_shape=jax.ShapeDtypeStruct((K, D), jnp.float32),
        scratch_shapes=[
            pltpu.VMEM((256, 256), jnp.float32),
            pltpu.VMEM((_LB * M, D), jnp.float32),
        ],
    )(lib_h, lib_adj, wn, wm, wr)


_GB = 128


def _gnn1_body(h_ref, a_ref, cond_ref, wn_ref, wm_ref, wr_ref, wca_ref,
               wcb_ref, wi0_ref, wi1a_ref, gv1_ref, t1_ref, bd_ref, msg_ref):
    @pl.when(pl.program_id(0) == 0)
    def _z():
        bd_ref[...] = jnp.zeros((256, 256), jnp.float32)

    a = a_ref[...].astype(jnp.float32)
    h2 = h_ref[...].reshape(_GB * V, FIN)
    x = jnp.maximum(jnp.dot(h2, wn_ref[...], preferred_element_type=jnp.float32), 0.0)
    z = jnp.dot(x, wm_ref[...], preferred_element_type=jnp.float32)
    npack = _GB * V // 256
    for p in range(npack):
        for g in range(8):
            bd_ref[V * g:V * (g + 1), V * g:V * (g + 1)] = a[8 * p + g]
        msg_ref[256 * p:256 * (p + 1), :] = jnp.dot(
            bd_ref[...], z[256 * p:256 * (p + 1), :],
            preferred_element_type=jnp.float32)
    hh = jnp.maximum(x + msg_ref[...], 0.0)

    xi = jnp.dot(hh, wi0_ref[...], preferred_element_type=jnp.float32)
    for p in range(npack):
        for g in range(8):
            bd_ref[V * g:V * (g + 1), V * g:V * (g + 1)] = a[8 * p + g]
        msg_ref[256 * p:256 * (p + 1), :] = jnp.dot(
            bd_ref[...], xi[256 * p:256 * (p + 1), :],
            preferred_element_type=jnp.float32)
    xe = jnp.maximum(msg_ref[...], 0.0)
    t1 = jnp.dot(xe, wi1a_ref[...], preferred_element_type=jnp.float32)
    t1_ref[...] = t1.reshape(_GB, V, D)

    gs = jnp.sum(hh.reshape(_GB, V, D), axis=1)
    gr = jnp.dot(gs, wr_ref[...], preferred_element_type=jnp.float32)
    gv1_ref[...] = jnp.maximum(
        jnp.dot(gr, wca_ref[...], preferred_element_type=jnp.float32)
        + jnp.dot(cond_ref[...], wcb_ref[...], preferred_element_type=jnp.float32),
        0.0)


def _gnn1(h, adj, cond, wn, wm, wr, wcond, wi0, wi1a):
    return pl.pallas_call(
        _gnn1_body,
        grid=(N // _GB,),
        in_specs=[
            pl.BlockSpec((_GB, V, FIN), lambda i: (i, 0, 0)),
            pl.BlockSpec((_GB, V, V), lambda i: (i, 0, 0)),
            pl.BlockSpec((_GB, FC), lambda i: (i, 0)),
            pl.BlockSpec((FIN, D), lambda i: (0, 0)),
            pl.BlockSpec((D, D), lambda i: (0, 0)),
            pl.BlockSpec((D, D), lambda i: (0, 0)),
            pl.BlockSpec((D, D), lambda i: (0, 0)),
            pl.BlockSpec((FC, D), lambda i: (0, 0)),
            pl.BlockSpec((D, D), lambda i: (0, 0)),
            pl.BlockSpec((D, D), lambda i: (0, 0)),
        ],
        out_specs=[
            pl.BlockSpec((_GB, D), lambda i: (i, 0)),
            pl.BlockSpec((_GB, V, D), lambda i: (i, 0, 0)),
        ],
        out_shape=[
            jax.ShapeDtypeStruct((N, D), jnp.float32),
            jax.ShapeDtypeStruct((N, V, D), jnp.float32),
        ],
        scratch_shapes=[
            pltpu.VMEM((256, 256), jnp.float32),
            pltpu.VMEM((_GB * V, D), jnp.float32),
        ],
    )(h, adj, cond, wn, wm, wr, wcond[:D], wcond[D:], wi0, wi1a)


_NW = 32
_BSEL = 3 * N
_BPW = _BSEL // _NW


def _sc_gather(table, idx):
    mesh = plsc.VectorSubcoreMesh(core_axis_name="c", subcore_axis_name="s")

    @functools.partial(
        pl.kernel, mesh=mesh,
        out_type=jax.ShapeDtypeStruct((_BSEL, D), jnp.float32),
        scratch_types=[
            pltpu.VMEM((_BPW,), jnp.int32),
            pltpu.VMEM((_BPW, D), jnp.float32),
            pltpu.SemaphoreType.DMA,
        ],
    )
    def gather_k(table_hbm, idx_hbm, out_hbm, idx_v, rows_v, sem):
        wid = lax.axis_index("s") * 2 + lax.axis_index("c")
        base = wid * _BPW
        pltpu.sync_copy(idx_hbm.at[pl.ds(base, _BPW)], idx_v)
        pltpu.async_copy(table_hbm.at[idx_v], rows_v, sem).wait()
        pltpu.sync_copy(rows_v, out_hbm.at[pl.ds(base, _BPW)])

    return gather_k(table, idx)


_FB = 256


def _loss_body(g1_ref, gp_ref, gn0_ref, gn1_ref, t1_ref, cond_ref, yf_ref,
               yi_ref, wb_ref, w2_ref, wt_ref, bt_ref, out_ref):
    i = pl.program_id(0)

    @pl.when(i == 0)
    def _z():
        out_ref[...] = jnp.zeros((1, 128), jnp.float32)

    g1 = g1_ref[...]
    gp = gp_ref[...]

    pt = 1.0 / (1.0 + jnp.exp(-(jnp.sum(g1 * wt_ref[...], axis=1, keepdims=True)
                                + bt_ref[0, 0])))
    yt = (yf_ref[...] == -1).astype(jnp.float32)
    s_term = jnp.sum(-(yt * jnp.log(pt + EPS)
                       + (1.0 - yt) * jnp.log(1.0 - pt)))

    pp = 1.0 / (1.0 + jnp.exp(-jnp.sum(g1 * gp, axis=1, keepdims=True)))
    s_p = jnp.sum(-jnp.log(pp + EPS))

    s_n = jnp.float32(0.0)
    for gn_ref in (gn0_ref, gn1_ref):
        pn = 1.0 / (1.0 + jnp.exp(-jnp.sum(g1 * gn_ref[...], axis=1, keepdims=True)))
        s_n = s_n + jnp.sum(-jnp.log(1.0 - pn))

    t2 = (jnp.dot(g1, wb_ref[0:D], preferred_element_type=jnp.float32)
          + jnp.dot(gp, wb_ref[D:2 * D], preferred_element_type=jnp.float32)
          + jnp.dot(cond_ref[...], wb_ref[2 * D:2 * D + FC],
                    preferred_element_type=jnp.float32))
    u = jnp.maximum(t1_ref[...] + t2[:, None, :], 0.0)
    logits = jnp.sum(u * w2_ref[0][None, None, :], axis=-1)
    mx = jnp.max(logits, axis=-1, keepdims=True)
    lse = mx + jnp.log(jnp.sum(jnp.exp(logits - mx), axis=-1, keepdims=True))
    io = lax.broadcasted_iota(jnp.int32, (_FB, V), 1)
    ly = jnp.sum(jnp.where(io == yi_ref[...], logits, 0.0), axis=-1,
                 keepdims=True)
    s_idx = jnp.sum(lse - ly)

    io128 = lax.broadcasted_iota(jnp.int32, (1, 128), 1)
    contrib = (jnp.where(io128 == 0, s_term, 0.0)
               + jnp.where(io128 == 1, s_p, 0.0)
               + jnp.where(io128 == 2, s_n, 0.0)
               + jnp.where(io128 == 3, s_idx, 0.0))
    acc = out_ref[...] + contrib

    @pl.when(i < pl.num_programs(0) - 1)
    def _store():
        out_ref[...] = acc

    @pl.when(i == pl.num_programs(0) - 1)
    def _final():
        scale = jnp.where(io128 == 2, 1.0 / (NSAMPLE * N), 1.0 / N)
        out_ref[...] = acc * scale.astype(jnp.float32)


def _losses(gv1, gvp, gvn0, gvn1, t1, cond, y_fid, y_idx, wi1b, w_idx2,
            w_term, b_term):
    res = pl.pallas_call(
        _loss_body,
        grid=(N // _FB,),
        in_specs=[
            pl.BlockSpec((_FB, D), lambda i: (i, 0)),
            pl.BlockSpec((_FB, D), lambda i: (i, 0)),
            pl.BlockSpec((_FB, D), lambda i: (i, 0)),
            pl.BlockSpec((_FB, D), lambda i: (i, 0)),
            pl.BlockSpec((_FB, V, D), lambda i: (i, 0, 0)),
            pl.BlockSpec((_FB, FC), lambda i: (i, 0)),
            pl.BlockSpec((_FB, 1), lambda i: (i, 0)),
            pl.BlockSpec((_FB, 1), lambda i: (i, 0)),
            pl.BlockSpec((2 * D + FC, D), lambda i: (0, 0)),
            pl.BlockSpec((1, D), lambda i: (0, 0)),
            pl.BlockSpec((1, D), lambda i: (0, 0)),
            pl.BlockSpec((1, 1), lambda i: (0, 0)),
        ],
        out_specs=pl.BlockSpec((1, 128), lambda i: (0, 0)),
        out_shape=jax.ShapeDtypeStruct((1, 128), jnp.float32),
    )(gv1, gvp, gvn0, gvn1, t1, cond, y_fid.reshape(N, 1),
      y_idx.reshape(N, 1), wi1b, w_idx2.reshape(1, D),
      w_term.reshape(1, D), b_term.reshape(1, 1))
    return res[0, 0:4]


def kernel(h, adj, cond, y_fid, y_idx, lib_h, lib_adj, lib_freq, W_node1,
           W_msg1, W_read1, W_cond, w_term, b_term, W_node2, W_msg2, W_read2,
           W_idx0, W_idx1, w_idx2):
    y_fid_c = jnp.maximum(y_fid.astype(jnp.int32), 0)

    yn0, yn1 = _sample_negatives(lib_freq, y_fid_c)
    gv2_all = _lib_embed(lib_h, lib_adj, W_node2, W_msg2, W_read2)
    gv1, t1 = _gnn1(h, adj, cond, W_node1, W_msg1, W_read1, W_cond, W_idx0,
                    W_idx1[0:D])

    idx_all = jnp.concatenate([y_fid_c, yn0, yn1])
    sel = _sc_gather(gv2_all, idx_all)
    gvp, gvn0, gvn1 = sel[0:N], sel[N:2 * N], sel[2 * N:3 * N]

    return _losses(gv1, gvp, gvn0, gvn1, t1, cond, y_fid, y_idx,
                   W_idx1[D:3 * D + FC], w_idx2, w_term, b_term)
